# Initial kernel scaffold; baseline (speedup 1.0000x reference)
import jax
import jax.numpy as jnp
from jax.experimental import pallas as pl


def kernel(x, edge_index, loop_weight):
    raise NotImplementedError("write your pallas kernel here")


# speedup vs baseline: 8.1807x; 8.1807x over previous
import functools

import jax
import jax.numpy as jnp
from jax import lax
from jax.experimental import pallas as pl
from jax.experimental.pallas import tpu as pltpu
from jax.experimental.pallas import tpu_sc as plsc

_N = 10000
_D = 128
_DH = _D // 2
_E = 320000
_EPS = 1e-5

_NC = 2
_NS = 16
_CHUNK = 128
_CHUNKS = 160
_E_PAD = _NS * _CHUNKS * _CHUNK
_PAD_ROWS = 16
_NP = _N + _PAD_ROWS


def _sc_halves(xh, src, dst, zeros):
    mesh = plsc.VectorSubcoreMesh(core_axis_name="c", subcore_axis_name="s")

    @functools.partial(
        pl.kernel,
        out_type=jax.ShapeDtypeStruct((_NC, _N, _DH), jnp.float32),
        mesh=mesh,
        compiler_params=pltpu.CompilerParams(use_tc_tiling_on_sc=False),
        scratch_types=[
            pltpu.VMEM((_CHUNKS, _CHUNK), jnp.int32),
            pltpu.VMEM((_CHUNKS, _CHUNK), jnp.int32),
            pltpu.VMEM((_CHUNK, _DH), jnp.float32),
            pltpu.VMEM((_CHUNK, _DH), jnp.float32),
            pltpu.VMEM_SHARED((_NP, _DH), jnp.float32),
            pltpu.SemaphoreType.DMA,
            pltpu.SemaphoreType.DMA,
        ],
    )
    def k(x_hbm, src_hbm, dst_hbm, z_hbm, out_hbm,
          src_v, dst_v, buf0, buf1, acc, sem0, sem1):
        c = lax.axis_index("c")
        s = lax.axis_index("s")

        pltpu.sync_copy(src_hbm.at[s], src_v)
        pltpu.sync_copy(dst_hbm.at[s], dst_v)

        @pl.when(s < _NS - 1)
        def _():
            pltpu.sync_copy(z_hbm.at[pl.ds(s * 632, 632)],
                            acc.at[pl.ds(s * 632, 632)])

        @pl.when(s == _NS - 1)
        def _():
            pltpu.sync_copy(z_hbm.at[pl.ds(15 * 632, _NP - 15 * 632)],
                            acc.at[pl.ds(15 * 632, _NP - 15 * 632)])

        plsc.subcore_barrier()
        xc = x_hbm.at[c]

        def body(i, carry):
            j0 = 2 * i
            j1 = j0 + 1
            cp0 = pltpu.async_copy(xc.at[src_v.at[j0]], buf0, sem0)
            cp1 = pltpu.async_copy(xc.at[src_v.at[j1]], buf1, sem1)
            cp0.wait()
            pltpu.sync_copy(buf0, acc.at[dst_v.at[j0]], add=True)
            cp1.wait()
            pltpu.sync_copy(buf1, acc.at[dst_v.at[j1]], add=True)
            return carry

        lax.fori_loop(0, _CHUNKS // 2, body, 0)
        plsc.subcore_barrier()

        @pl.when(s < _NS - 1)
        def _():
            pltpu.sync_copy(acc.at[pl.ds(s * 632, 632)],
                            out_hbm.at[c, pl.ds(s * 632, 632)])

        @pl.when(s == _NS - 1)
        def _():
            pltpu.sync_copy(acc.at[pl.ds(15 * 632, _N - 15 * 632)],
                            out_hbm.at[c, pl.ds(15 * 632, _N - 15 * 632)])

    return k(xh, src, dst, zeros)


def _tc_finish(x, w, parts):
    blk = 1000

    def body(x_ref, w_ref, p0_ref, p1_ref, o_ref):
        h = jnp.dot(x_ref[...], w_ref[...], preferred_element_type=jnp.float32)
        h = h + jnp.concatenate([p0_ref[0], p1_ref[0]], axis=-1)
        mu = jnp.mean(h, axis=-1, keepdims=True)
        d = h - mu
        var = jnp.mean(d * d, axis=-1, keepdims=True)
        o_ref[...] = d * lax.rsqrt(var + _EPS)

    return pl.pallas_call(
        body,
        grid=(_N // blk,),
        in_specs=[
            pl.BlockSpec((blk, _D), lambda i: (i, 0)),
            pl.BlockSpec((_D, _D), lambda i: (0, 0)),
            pl.BlockSpec((1, blk, _DH), lambda i: (0, i, 0)),
            pl.BlockSpec((1, blk, _DH), lambda i: (1, i, 0)),
        ],
        out_specs=pl.BlockSpec((blk, _D), lambda i: (i, 0)),
        out_shape=jax.ShapeDtypeStruct((_N, _D), jnp.float32),
    )(x, w, parts, parts)


def kernel(x, edge_index, loop_weight):
    src = edge_index[0]
    dst = edge_index[1]
    pad = _E_PAD - _E
    ar = jnp.arange(pad, dtype=jnp.int32)
    src_p = jnp.concatenate([src, (ar * 13) % _N]).reshape(_NS, _CHUNKS, _CHUNK)
    dst_p = jnp.concatenate([dst, _N + (ar % _PAD_ROWS)]).reshape(_NS, _CHUNKS, _CHUNK)
    xh = jnp.stack([x[:, :_DH], x[:, _DH:]])
    zeros = jnp.zeros((_NP, _DH), jnp.float32)
    parts = _sc_halves(xh, src_p, dst_p, zeros)
    return _tc_finish(x, loop_weight, parts)
